# speedup vs baseline: 1.2663x; 1.2663x over previous
import functools

import jax
import jax.numpy as jnp
from jax import lax
from jax.experimental import pallas as pl
from jax.experimental.pallas import tpu as pltpu
from jax.experimental.pallas import tpu_sc as plsc

L_CTX = 2048
D = 128
B = 4
N = B * L_CTX
NC = 2
NS = 16
NW = NC * NS
PER_W = N // NW
GCH = 128
NG = PER_W // GCH
LANES = 16

_mesh = plsc.VectorSubcoreMesh(core_axis_name="c", subcore_axis_name="s")


@functools.partial(
    pl.kernel,
    out_type=jax.ShapeDtypeStruct((N, D), jnp.float32),
    mesh=_mesh,
    scratch_types=[
        pltpu.VMEM((NG, GCH), jnp.int32),
        pltpu.VMEM((PER_W, D), jnp.float32),
        pltpu.VMEM((PER_W, D), jnp.float32),
        pltpu.SemaphoreType.DMA,
    ],
)
def _emb_lookup(idx_hbm, tok_hbm, pos_hbm, out_hbm, idx_v, rows_v, pos_v, sem):
    c = lax.axis_index("c")
    s = lax.axis_index("s")
    wid = s * NC + c
    base = wid * PER_W
    pos_base = lax.rem(base, L_CTX)

    pltpu.sync_copy(idx_hbm.at[wid], idx_v)

    copies = [
        pltpu.async_copy(
            tok_hbm.at[idx_v.at[g]], rows_v.at[pl.ds(g * GCH, GCH)], sem
        )
        for g in range(NG)
    ]
    pltpu.sync_copy(pos_hbm.at[pl.ds(pos_base, PER_W)], pos_v)
    for cp in copies:
        cp.wait()

    def row_body(j, carry):
        for k in range(D // LANES):
            sl = pl.ds(k * LANES, LANES)
            rows_v[j, sl] = rows_v[j, sl] + pos_v[j, sl]
        return carry

    lax.fori_loop(0, PER_W, row_body, 0)

    pltpu.sync_copy(rows_v, out_hbm.at[pl.ds(base, PER_W)])


def kernel(inputs, token_table, position_table):
    idx = inputs.astype(jnp.int32).reshape(NW, NG, GCH)
    out = _emb_lookup(idx, token_table, position_table)
    return out.reshape(B, L_CTX, D)


# speedup vs baseline: 1.3479x; 1.0644x over previous
import functools

import jax
import jax.numpy as jnp
from jax import lax
from jax.experimental import pallas as pl
from jax.experimental.pallas import tpu as pltpu
from jax.experimental.pallas import tpu_sc as plsc

L_CTX = 2048
D = 128
B = 4
N = B * L_CTX
NC = 2
NS = 16
NW = NC * NS
PER_W = N // NW
GCH = 128
NG = PER_W // GCH
LANES = 16

_mesh = plsc.VectorSubcoreMesh(core_axis_name="c", subcore_axis_name="s")


@functools.partial(
    pl.kernel,
    out_type=jax.ShapeDtypeStruct((N, D), jnp.float32),
    mesh=_mesh,
    scratch_types=[
        pltpu.VMEM((NG, GCH), jnp.int32),
        pltpu.VMEM((PER_W, D), jnp.float32),
        pltpu.VMEM((PER_W, D), jnp.float32),
        pltpu.SemaphoreType.DMA,
    ],
)
def _emb_lookup(idx_hbm, tok_hbm, pos_hbm, out_hbm, idx_v, rows_v, pos_v, sem):
    c = lax.axis_index("c")
    s = lax.axis_index("s")
    wid = s * NC + c
    base = wid * PER_W
    pos_base = lax.rem(base, L_CTX)

    pltpu.sync_copy(idx_hbm.at[wid], idx_v)
    pltpu.sync_copy(pos_hbm.at[pl.ds(pos_base, PER_W)], rows_v)

    copies = [
        pltpu.async_copy(
            tok_hbm.at[idx_v.at[g]], rows_v.at[pl.ds(g * GCH, GCH)], sem,
            add=True,
        )
        for g in range(NG)
    ]
    for cp in copies:
        cp.wait()

    pltpu.sync_copy(rows_v, out_hbm.at[pl.ds(base, PER_W)])


def kernel(inputs, token_table, position_table):
    idx = inputs.astype(jnp.int32).reshape(NW, NG, GCH)
    out = _emb_lookup(idx, token_table, position_table)
    return out.reshape(B, L_CTX, D)


# speedup vs baseline: 1.3751x; 1.0201x over previous
import functools

import jax
import jax.numpy as jnp
from jax import lax
from jax.experimental import pallas as pl
from jax.experimental.pallas import tpu as pltpu
from jax.experimental.pallas import tpu_sc as plsc

L_CTX = 2048
D = 128
B = 4
N = B * L_CTX
NC = 2
NS = 16
NW = NC * NS
P_W = L_CTX // NW
LANES = 16

_mesh = plsc.VectorSubcoreMesh(core_axis_name="c", subcore_axis_name="s")


@functools.partial(
    pl.kernel,
    out_type=jax.ShapeDtypeStruct((N, D), jnp.float32),
    mesh=_mesh,
    scratch_types=[
        pltpu.VMEM((B, P_W), jnp.int32),
        pltpu.VMEM((P_W, D), jnp.float32),
        pltpu.VMEM((B * P_W, D), jnp.float32),
        pltpu.SemaphoreType.DMA,
        pltpu.SemaphoreType.DMA((B,)),
        pltpu.SemaphoreType.DMA((B,)),
    ],
)
def _emb_lookup(idx_hbm, tok_hbm, pos_hbm, out_hbm,
                idx_v, pos_v, rows_v, sem_p, sem_g, sem_w):
    c = lax.axis_index("c")
    s = lax.axis_index("s")
    wid = s * NC + c
    p0 = wid * P_W

    pos_cp = pltpu.async_copy(pos_hbm.at[pl.ds(p0, P_W)], pos_v, sem_p)
    for b in range(B):
        pltpu.sync_copy(idx_hbm.at[b, pl.ds(p0, P_W)], idx_v.at[b])
    pos_cp.wait()

    gathers = []
    for b in range(B):
        def rep_body(j, carry, _b=b):
            for k in range(D // LANES):
                sl = pl.ds(k * LANES, LANES)
                rows_v[_b * P_W + j, sl] = pos_v[j, sl]
            return carry

        lax.fori_loop(0, P_W, rep_body, 0)
        gathers.append(
            pltpu.async_copy(
                tok_hbm.at[idx_v.at[b]],
                rows_v.at[pl.ds(b * P_W, P_W)],
                sem_g.at[b],
                add=True,
            )
        )

    writes = []
    for b in range(B):
        gathers[b].wait()
        writes.append(
            pltpu.async_copy(
                rows_v.at[pl.ds(b * P_W, P_W)],
                out_hbm.at[pl.ds(b * L_CTX + p0, P_W)],
                sem_w.at[b],
            )
        )
    for w in writes:
        w.wait()


def kernel(inputs, token_table, position_table):
    out = _emb_lookup(inputs.astype(jnp.int32), token_table, position_table)
    return out.reshape(B, L_CTX, D)


# speedup vs baseline: 1.4174x; 1.0308x over previous
import functools

import jax
import jax.numpy as jnp
from jax import lax
from jax.experimental import pallas as pl
from jax.experimental.pallas import tpu as pltpu
from jax.experimental.pallas import tpu_sc as plsc

L_CTX = 2048
D = 128
B = 4
N = B * L_CTX
NC = 2
NS = 16
NW = NC * NS
P_W = L_CTX // NW
LANES = 16

_mesh = plsc.VectorSubcoreMesh(core_axis_name="c", subcore_axis_name="s")


@functools.partial(
    pl.kernel,
    out_type=jax.ShapeDtypeStruct((N, D), jnp.float32),
    mesh=_mesh,
    scratch_types=[
        pltpu.VMEM((B, P_W), jnp.int32),
        pltpu.VMEM((P_W, D), jnp.float32),
        pltpu.VMEM((B * P_W, D), jnp.float32),
        pltpu.SemaphoreType.DMA((B,)),
        pltpu.SemaphoreType.DMA,
        pltpu.SemaphoreType.DMA((B,)),
        pltpu.SemaphoreType.DMA((B,)),
    ],
)
def _emb_lookup(idx_hbm, tok_hbm, pos_hbm, out_hbm,
                idx_v, pos_v, rows_v, sem_i, sem_p, sem_g, sem_w):
    c = lax.axis_index("c")
    s = lax.axis_index("s")
    wid = s * NC + c
    p0 = wid * P_W

    idx_cps = [
        pltpu.async_copy(
            idx_hbm.at[b, pl.ds(p0, P_W)], idx_v.at[b], sem_i.at[b]
        )
        for b in range(B)
    ]
    pos_cp = pltpu.async_copy(pos_hbm.at[pl.ds(p0, P_W)], pos_v, sem_p)
    pos_cp.wait()

    gathers = []
    for b in range(B):
        def rep_body(j, carry, _b=b):
            for k in range(D // LANES):
                sl = pl.ds(k * LANES, LANES)
                rows_v[_b * P_W + j, sl] = pos_v[j, sl]
            return carry

        lax.fori_loop(0, P_W, rep_body, 0)
        idx_cps[b].wait()
        gathers.append(
            pltpu.async_copy(
                tok_hbm.at[idx_v.at[b]],
                rows_v.at[pl.ds(b * P_W, P_W)],
                sem_g.at[b],
                add=True,
            )
        )

    writes = []
    for b in range(B):
        gathers[b].wait()
        writes.append(
            pltpu.async_copy(
                rows_v.at[pl.ds(b * P_W, P_W)],
                out_hbm.at[pl.ds(b * L_CTX + p0, P_W)],
                sem_w.at[b],
            )
        )
    for w in writes:
        w.wait()


def kernel(inputs, token_table, position_table):
    out = _emb_lookup(inputs.astype(jnp.int32), token_table, position_table)
    return out.reshape(B, L_CTX, D)


# speedup vs baseline: 1.4486x; 1.0220x over previous
import functools

import jax
import jax.numpy as jnp
from jax import lax
from jax.experimental import pallas as pl
from jax.experimental.pallas import tpu as pltpu
from jax.experimental.pallas import tpu_sc as plsc

L_CTX = 2048
D = 128
B = 4
N = B * L_CTX
NC = 2
NS = 16
NW = NC * NS
P_W = L_CTX // NW
W_CH = 32
N_WCH = P_W // W_CH
LANES = 16

_mesh = plsc.VectorSubcoreMesh(core_axis_name="c", subcore_axis_name="s")


@functools.partial(
    pl.kernel,
    out_type=jax.ShapeDtypeStruct((N, D), jnp.float32),
    mesh=_mesh,
    scratch_types=[
        pltpu.VMEM((B, P_W), jnp.int32),
        pltpu.VMEM((P_W, D), jnp.float32),
        pltpu.VMEM((P_W, D), jnp.float32),
        pltpu.VMEM((B * P_W, D), jnp.float32),
        pltpu.SemaphoreType.DMA((B,)),
        pltpu.SemaphoreType.DMA,
        pltpu.SemaphoreType.DMA((B,)),
        pltpu.SemaphoreType.DMA((B * N_WCH,)),
    ],
)
def _emb_lookup(idx_hbm, tok_hbm, pos_hbm, out_hbm,
                idx_v, pos_v, g0_v, rows_v, sem_i, sem_p, sem_g, sem_w):
    c = lax.axis_index("c")
    s = lax.axis_index("s")
    wid = s * NC + c
    p0 = wid * P_W

    idx_cps = [
        pltpu.async_copy(
            idx_hbm.at[b, pl.ds(p0, P_W)], idx_v.at[b], sem_i.at[b]
        )
        for b in range(B)
    ]
    pos_cp = pltpu.async_copy(pos_hbm.at[pl.ds(p0, P_W)], pos_v, sem_p)

    idx_cps[0].wait()
    gathers = [
        pltpu.async_copy(tok_hbm.at[idx_v.at[0]], g0_v, sem_g.at[0])
    ]
    pos_cp.wait()

    for b in range(1, B):
        def rep_body(j, carry, _b=b):
            for k in range(D // LANES):
                sl = pl.ds(k * LANES, LANES)
                rows_v[_b * P_W + j, sl] = pos_v[j, sl]
            return carry

        lax.fori_loop(0, P_W, rep_body, 0)
        idx_cps[b].wait()
        gathers.append(
            pltpu.async_copy(
                tok_hbm.at[idx_v.at[b]],
                rows_v.at[pl.ds(b * P_W, P_W)],
                sem_g.at[b],
                add=True,
            )
        )

    writes = []

    def emit_writes(src, b):
        for h in range(N_WCH):
            writes.append(
                pltpu.async_copy(
                    src.at[pl.ds(h * W_CH, W_CH)],
                    out_hbm.at[pl.ds(b * L_CTX + p0 + h * W_CH, W_CH)],
                    sem_w.at[b * N_WCH + h],
                )
            )

    gathers[0].wait()

    def add_body(j, carry):
        for k in range(D // LANES):
            sl = pl.ds(k * LANES, LANES)
            g0_v[j, sl] = g0_v[j, sl] + pos_v[j, sl]
        return carry

    lax.fori_loop(0, P_W, add_body, 0)
    emit_writes(g0_v, 0)

    for b in range(1, B):
        gathers[b].wait()
        emit_writes(rows_v.at[pl.ds(b * P_W, P_W)], b)

    for w in writes:
        w.wait()


def kernel(inputs, token_table, position_table):
    out = _emb_lookup(inputs.astype(jnp.int32), token_table, position_table)
    return out.reshape(B, L_CTX, D)
